# baseline (speedup 1.0000x reference)
import functools

import jax
import jax.numpy as jnp
from jax import lax
from jax.experimental import pallas as pl
from jax.experimental.pallas import tpu as pltpu
from jax.experimental.pallas import tpu_sc as plsc

_E_ROWS = 160000
_E_COLS = 16
_NC = 2
_NS = 16
_ROWS_PER_WORKER = _E_ROWS // (_NC * _NS)


def _x_copy_body(x_ref, xo_ref):
    xo_ref[...] = x_ref[...]


def _copy_x(x):
    return pl.pallas_call(
        _x_copy_body,
        grid=(5,),
        in_specs=[pl.BlockSpec((2000, 256), lambda i: (i, 0))],
        out_specs=pl.BlockSpec((2000, 256), lambda i: (i, 0)),
        out_shape=jax.ShapeDtypeStruct(x.shape, x.dtype),
    )(x)


@functools.partial(
    pl.kernel,
    out_type=jax.ShapeDtypeStruct((_E_ROWS, _E_COLS), jnp.float32),
    mesh=plsc.VectorSubcoreMesh(core_axis_name="c", subcore_axis_name="s"),
    scratch_types=[pltpu.VMEM((_ROWS_PER_WORKER, _E_COLS), jnp.float32)],
    compiler_params=pltpu.CompilerParams(use_tc_tiling_on_sc=False),
)
def _copy_e(e_hbm, out_hbm, buf):
    wid = lax.axis_index("s") * _NC + lax.axis_index("c")
    base = wid * _ROWS_PER_WORKER
    pltpu.sync_copy(e_hbm.at[pl.ds(base, _ROWS_PER_WORKER)], buf)
    pltpu.sync_copy(buf, out_hbm.at[pl.ds(base, _ROWS_PER_WORKER)])


def kernel(x, edge_index, edge_attr):
    del edge_index
    return (_copy_x(x), _copy_e(edge_attr))
